# baseline (speedup 1.0000x reference)
import functools
import jax
import jax.numpy as jnp
from jax import lax
from jax.experimental import pallas as pl
from jax.experimental.pallas import tpu as pltpu
from jax.experimental.pallas import tpu_sc as plsc

B = 8192
D = 1024
H = 256
E = 3
EP = 128
BM = 512

NC = 2
NS = 16
L = 16
NW = NC * NS
C = 8
WB = 3 * L

RT = 5120
B2 = B - RT


def _gate_kernel(cs_ref, na_ref, gamma_ref, beta_ref, w1a_ref, w1b_ref,
                 b1_ref, w2p_ref, b2p_ref, wts_ref, wb_ref):
    cs = cs_ref[...]
    mu = jnp.mean(cs, axis=1, keepdims=True)
    xc = cs - mu
    var = jnp.mean(xc * xc, axis=1, keepdims=True)
    ns = xc * jax.lax.rsqrt(var + 1e-5) * gamma_ref[...] + beta_ref[...]
    h = (jnp.dot(ns.astype(jnp.bfloat16), w1a_ref[...],
                 preferred_element_type=jnp.float32)
         + jnp.dot(na_ref[...].astype(jnp.bfloat16), w1b_ref[...],
                   preferred_element_type=jnp.float32)
         + b1_ref[...])
    h = 0.5 * h * (1.0 + jax.lax.erf(h * 0.7071067811865476))
    logits = jnp.dot(h, w2p_ref[...], preferred_element_type=jnp.float32) + b2p_ref[...]
    m = jnp.max(logits, axis=1, keepdims=True)
    ex = jnp.exp(logits - m)
    w = ex / jnp.sum(ex, axis=1, keepdims=True)
    wts_ref[...] = w
    wb_ref[...] = jnp.concatenate(
        [jnp.broadcast_to(w[:, 0:1], (BM, L)),
         jnp.broadcast_to(w[:, 1:2], (BM, L)),
         jnp.broadcast_to(w[:, 2:3], (BM, L))], axis=1)


def _gate(cs, na, gamma, beta, w1a, w1b, b1r, w2p, b2p):
    grid = (B // BM,)
    row = lambda i: (i, 0)
    rep = lambda i: (0, 0)
    return pl.pallas_call(
        _gate_kernel,
        grid=grid,
        in_specs=[
            pl.BlockSpec((BM, D), row),
            pl.BlockSpec((BM, D), row),
            pl.BlockSpec((1, D), rep),
            pl.BlockSpec((1, D), rep),
            pl.BlockSpec((D, H), rep),
            pl.BlockSpec((D, H), rep),
            pl.BlockSpec((1, H), rep),
            pl.BlockSpec((H, EP), rep),
            pl.BlockSpec((1, EP), rep),
        ],
        out_specs=[
            pl.BlockSpec((BM, EP), row),
            pl.BlockSpec((BM, WB), row),
        ],
        out_shape=[
            jax.ShapeDtypeStruct((B, EP), jnp.float32),
            jax.ShapeDtypeStruct((B, WB), jnp.float32),
        ],
    )(cs, na, gamma, beta, w1a, w1b, b1r, w2p, b2p)


def _tc_combine_kernel(e0_ref, e1_ref, e2_ref, w_ref, out_ref):
    w = w_ref[...]
    out_ref[...] = (w[:, 0:1] * e0_ref[...]
                    + w[:, 1:2] * e1_ref[...]
                    + w[:, 2:3] * e2_ref[...])


def _tc_combine(e0, e1, e2, wts):
    grid = (RT // BM,)
    row = lambda i: (i, 0)
    return pl.pallas_call(
        _tc_combine_kernel,
        grid=grid,
        in_specs=[
            pl.BlockSpec((BM, D), row),
            pl.BlockSpec((BM, D), row),
            pl.BlockSpec((BM, D), row),
            pl.BlockSpec((BM, EP), row),
        ],
        out_specs=pl.BlockSpec((BM, D), row),
        out_shape=jax.ShapeDtypeStruct((RT, D), jnp.float32),
    )(e0, e1, e2, wts)


_sc_mesh = plsc.VectorSubcoreMesh(core_axis_name="c", subcore_axis_name="s")
_RPW = B2 // NW
_NCH = _RPW // C


@functools.partial(
    pl.kernel,
    out_type=jax.ShapeDtypeStruct((B2, D), jnp.float32),
    mesh=_sc_mesh,
    scratch_types=[
        pltpu.VMEM((2, C, D), jnp.float32),
        pltpu.VMEM((2, C, D), jnp.float32),
        pltpu.VMEM((2, C, D), jnp.float32),
        pltpu.VMEM((2, C, WB), jnp.float32),
        pltpu.VMEM((2, C, D), jnp.float32),
        pltpu.SemaphoreType.DMA,
        pltpu.SemaphoreType.DMA,
        pltpu.SemaphoreType.DMA,
        pltpu.SemaphoreType.DMA,
    ],
)
def _sc_combine(e0_hbm, e1_hbm, e2_hbm, wb_hbm, out_hbm,
                e0v, e1v, e2v, wbv, outv, isem0, isem1, osem0, osem1):
    wid = lax.axis_index("s") * NC + lax.axis_index("c")
    base_in = RT + wid * _RPW
    base_out = wid * _RPW
    isems = (isem0, isem1)
    osems = (osem0, osem1)

    def in_copies(g, b):
        r0 = base_in + g * C
        sem = isems[b]
        return (pltpu.make_async_copy(e0_hbm.at[pl.ds(r0, C)], e0v.at[b], sem),
                pltpu.make_async_copy(e1_hbm.at[pl.ds(r0, C)], e1v.at[b], sem),
                pltpu.make_async_copy(e2_hbm.at[pl.ds(r0, C)], e2v.at[b], sem),
                pltpu.make_async_copy(wb_hbm.at[pl.ds(r0, C)], wbv.at[b], sem))

    def out_copy(g, b):
        r0 = base_out + g * C
        return pltpu.make_async_copy(outv.at[b], out_hbm.at[pl.ds(r0, C)],
                                     osems[b])

    def start_in(g, b):
        for cp in in_copies(g, b):
            cp.start()

    def wait_in(g, b):
        for cp in in_copies(g, b):
            cp.wait()

    def compute_chunk(b):
        def row_body(r, _):
            w0 = wbv[b, r, 0:L]
            w1 = wbv[b, r, L:2 * L]
            w2 = wbv[b, r, 2 * L:3 * L]

            @plsc.parallel_loop(0, D, step=L, unroll=8, carry=(w0, w1, w2))
            def _vecs(off, ws):
                a0, a1, a2 = ws
                sl = pl.ds(off, L)
                outv[b, r, sl] = (a0 * e0v[b, r, sl] + a1 * e1v[b, r, sl]
                                  + a2 * e2v[b, r, sl])
                return ws

            return 0

        lax.fori_loop(0, C, row_body, 0)

    start_in(0, 0)
    start_in(1, 1)

    def super_body(tt, _):
        for b in range(2):
            g = 2 * tt + b
            wait_in(g, b)

            @pl.when(tt > 0)
            def _():
                out_copy(g - 2, b).wait()

            compute_chunk(b)
            out_copy(g, b).start()

            @pl.when(tt < _NCH // 2 - 1)
            def _():
                start_in(g + 2, b)
        return 0

    lax.fori_loop(0, _NCH // 2, super_body, 0)
    out_copy(_NCH - 2, 0).wait()
    out_copy(_NCH - 1, 1).wait()


def kernel(current_state, neighbor_activity, expert_out_0, expert_out_1, expert_out_2, ln_gamma, ln_beta, W1, b1, W2, b2):
    gamma = ln_gamma.reshape(1, D)
    beta = ln_beta.reshape(1, D)
    w1a = W1[:D].astype(jnp.bfloat16)
    w1b = W1[D:].astype(jnp.bfloat16)
    b1r = b1.reshape(1, H)
    w2p = jnp.zeros((H, EP), jnp.float32).at[:, :E].set(W2)
    b2p = jnp.full((1, EP), -1e30, jnp.float32).at[0, :E].set(b2)

    wtsp, wb = _gate(current_state, neighbor_activity, gamma, beta,
                     w1a, w1b, b1r, w2p, b2p)
    bot = _sc_combine(expert_out_0, expert_out_1, expert_out_2, wb)
    top = _tc_combine(expert_out_0, expert_out_1, expert_out_2, wtsp)
    out = jnp.concatenate([top, bot], axis=0)
    return out, wtsp[:, :E]


# speedup vs baseline: 1.7130x; 1.7130x over previous
import jax
import jax.numpy as jnp
from jax.experimental import pallas as pl

B = 8192
D = 1024
H = 256
E = 3
EP = 128
BM = 512


def _fused_kernel(cs_ref, na_ref, e0_ref, e1_ref, e2_ref, gamma_ref, beta_ref,
                  w1a_ref, w1b_ref, b1_ref, w2p_ref, b2p_ref,
                  out_ref, wts_ref):
    cs = cs_ref[...]
    mu = jnp.mean(cs, axis=1, keepdims=True)
    xc = cs - mu
    var = jnp.mean(xc * xc, axis=1, keepdims=True)
    ns = xc * jax.lax.rsqrt(var + 1e-5) * gamma_ref[...] + beta_ref[...]
    h = (jnp.dot(ns.astype(jnp.bfloat16), w1a_ref[...],
                 preferred_element_type=jnp.float32)
         + jnp.dot(na_ref[...].astype(jnp.bfloat16), w1b_ref[...],
                   preferred_element_type=jnp.float32)
         + b1_ref[...])
    h = 0.5 * h * (1.0 + jax.lax.erf(h * 0.7071067811865476))
    logits = jnp.dot(h, w2p_ref[...], preferred_element_type=jnp.float32) + b2p_ref[...]
    m = jnp.max(logits, axis=1, keepdims=True)
    ex = jnp.exp(logits - m)
    w = ex / jnp.sum(ex, axis=1, keepdims=True)
    wts_ref[...] = w
    out_ref[...] = (w[:, 0:1] * e0_ref[...]
                    + w[:, 1:2] * e1_ref[...]
                    + w[:, 2:3] * e2_ref[...])


def kernel(current_state, neighbor_activity, expert_out_0, expert_out_1, expert_out_2, ln_gamma, ln_beta, W1, b1, W2, b2):
    gamma = ln_gamma.reshape(1, D)
    beta = ln_beta.reshape(1, D)
    w1a = W1[:D].astype(jnp.bfloat16)
    w1b = W1[D:].astype(jnp.bfloat16)
    b1r = b1.reshape(1, H)
    w2p = jnp.zeros((H, EP), jnp.float32).at[:, :E].set(W2)
    b2p = jnp.full((1, EP), -1e30, jnp.float32).at[0, :E].set(b2)

    grid = (B // BM,)
    row = lambda i: (i, 0)
    rep = lambda i: (0, 0)
    out, wts = pl.pallas_call(
        _fused_kernel,
        grid=grid,
        in_specs=[
            pl.BlockSpec((BM, D), row),
            pl.BlockSpec((BM, D), row),
            pl.BlockSpec((BM, D), row),
            pl.BlockSpec((BM, D), row),
            pl.BlockSpec((BM, D), row),
            pl.BlockSpec((1, D), rep),
            pl.BlockSpec((1, D), rep),
            pl.BlockSpec((D, H), rep),
            pl.BlockSpec((D, H), rep),
            pl.BlockSpec((1, H), rep),
            pl.BlockSpec((H, EP), rep),
            pl.BlockSpec((1, EP), rep),
        ],
        out_specs=[
            pl.BlockSpec((BM, D), row),
            pl.BlockSpec((BM, EP), row),
        ],
        out_shape=[
            jax.ShapeDtypeStruct((B, D), jnp.float32),
            jax.ShapeDtypeStruct((B, EP), jnp.float32),
        ],
    )(current_state, neighbor_activity, expert_out_0, expert_out_1,
      expert_out_2, gamma, beta, w1a, w1b, b1r, w2p, b2p)
    return out, wts[:, :E]
